# baseline (speedup 1.0000x reference)
import functools
import math

import jax
import jax.numpy as jnp
from jax import lax
from jax.experimental import pallas as pl
from jax.experimental.pallas import tpu as pltpu


def _rms(x, eps=1e-6):
    var = jnp.mean(x * x, axis=-1, keepdims=True)
    return x * lax.rsqrt(var + eps)


def _core(nb, p_ref, t_ref, o_ref, qn_ref, pn_ref):
    i = pl.program_id(0)
    d = p_ref.shape[-1]
    scale = 1.0 / math.sqrt(d)
    sl = lax.rem(i, 2)

    @pl.when(i > 0)
    def _emit():
        pv = 1 - sl
        qn_p = qn_ref[pv]
        pn_p = pn_ref[pv]
        score = lax.dot_general(
            qn_p, pn_p, (((1,), (1,)), ((), ())),
            preferred_element_type=jnp.float32) * scale
        m = jnp.max(score, axis=-1, keepdims=True)
        o_ref[0] = jnp.where(score == m, 1.0, 0.0).astype(jnp.float32)

    @pl.when(i < nb)
    def _compute():
        p = p_ref[0]
        t = t_ref[0]
        pn = _rms(p)
        tn = _rms(t)
        logits = lax.dot_general(
            pn, tn, (((1,), (1,)), ((), ())),
            preferred_element_type=jnp.float32) * scale
        attn = jax.nn.softmax(logits, axis=-1)
        q = lax.dot_general(
            attn, tn, (((1,), (0,)), ((), ())),
            preferred_element_type=jnp.float32)
        qn_ref[sl] = _rms(q)
        pn_ref[sl] = pn


def kernel(patches, task_tokens):
    b, n, d = patches.shape
    t = task_tokens.shape[1]
    return pl.pallas_call(
        functools.partial(_core, b),
        grid=(b + 1,),
        in_specs=[
            pl.BlockSpec((1, n, d), lambda i: (jnp.minimum(i, 15), 0, 0)),
            pl.BlockSpec((1, t, d), lambda i: (jnp.minimum(i, 15), 0, 0)),
        ],
        out_specs=pl.BlockSpec(
            (1, n, n), lambda i: (jnp.maximum(i - 1, 0), 0, 0)),
        out_shape=jax.ShapeDtypeStruct((b, n, n), jnp.float32),
        scratch_shapes=[
            pltpu.VMEM((2, n, d), jnp.float32),
            pltpu.VMEM((2, n, d), jnp.float32),
        ],
    )(patches, task_tokens)


# speedup vs baseline: 1.0833x; 1.0833x over previous
import math

import jax
import jax.numpy as jnp
from jax import lax
from jax.experimental import pallas as pl


def _rms(x, eps=1e-6):
    var = jnp.mean(x * x, axis=-1, keepdims=True)
    return x * lax.rsqrt(var + eps)


def _core(p_ref, t_ref, o_ref):
    p = p_ref[0]
    t = t_ref[0]
    d = p.shape[-1]
    scale = 1.0 / math.sqrt(d)

    pn = _rms(p)
    tn = _rms(t)
    logits = lax.dot_general(
        pn, tn, (((1,), (1,)), ((), ())),
        preferred_element_type=jnp.float32) * scale
    attn = jax.nn.softmax(logits, axis=-1)
    q = lax.dot_general(
        attn, tn, (((1,), (0,)), ((), ())),
        preferred_element_type=jnp.float32)
    qn = _rms(q)
    score = lax.dot_general(
        qn, pn, (((1,), (1,)), ((), ())),
        preferred_element_type=jnp.float32) * scale
    m = jnp.max(score, axis=-1, keepdims=True)
    o_ref[0] = jnp.where(score == m, 1.0, 0.0).astype(jnp.float32)


def kernel(patches, task_tokens):
    b, n, d = patches.shape
    t = task_tokens.shape[1]
    return pl.pallas_call(
        _core,
        grid=(b,),
        in_specs=[
            pl.BlockSpec((1, n, d), lambda i: (i, 0, 0)),
            pl.BlockSpec((1, t, d), lambda i: (i, 0, 0)),
        ],
        out_specs=pl.BlockSpec((1, n, n), lambda i: (i, 0, 0)),
        out_shape=jax.ShapeDtypeStruct((b, n, n), jnp.float32),
    )(patches, task_tokens)
